# baseline (speedup 1.0000x reference)
import functools

import jax
import jax.numpy as jnp
from jax.experimental import pallas as pl
from jax.experimental.pallas import tpu as pltpu

_NUM_EMB = 1024
_DIM = 64
_BATCH = 64
_SEQ = 576
_ROWS = _BATCH * _SEQ


def _vq_block_kernel(x_ref, w_ref, w2_ref, qst_ref, idx_ref, acc_ref):
    x = x_ref[0]
    w = w_ref[...]
    s = jax.lax.dot_general(x, w, (((1,), (1,)), ((), ())),
                            preferred_element_type=jnp.float32)
    x2 = jnp.sum(x * x, axis=1, keepdims=True)
    d = (x2 + w2_ref[...]) - 2.0 * s
    m = jnp.min(d, axis=1, keepdims=True)
    lane = jax.lax.broadcasted_iota(jnp.int32, (_SEQ, _NUM_EMB), 1)
    idx = jnp.min(jnp.where(d == m, lane, _NUM_EMB), axis=1)
    enc = (lane == idx[:, None]).astype(jnp.bfloat16)
    q = jax.lax.dot_general(enc, w.astype(jnp.bfloat16),
                            (((1,), (0,)), ((), ())),
                            preferred_element_type=jnp.float32)
    qst_ref[...] = (x + (q - x))[None]
    idx_ref[...] = idx[None, None, :]
    part = jnp.sum((q - x) ** 2)

    @pl.when(pl.program_id(0) == 0)
    def _init():
        acc_ref[...] = jnp.zeros_like(acc_ref)

    acc_ref[...] += part


@functools.partial(jax.jit, static_argnames=())
def kernel(inputs, W):
    w2 = jnp.sum(W ** 2, axis=1).reshape(1, _NUM_EMB)

    qst, idx, acc = pl.pallas_call(
        _vq_block_kernel,
        grid=(_BATCH,),
        in_specs=[
            pl.BlockSpec((1, _SEQ, _DIM), lambda i: (i, 0, 0)),
            pl.BlockSpec((_NUM_EMB, _DIM), lambda i: (0, 0)),
            pl.BlockSpec((1, _NUM_EMB), lambda i: (0, 0)),
        ],
        out_specs=[
            pl.BlockSpec((1, _SEQ, _DIM), lambda i: (i, 0, 0)),
            pl.BlockSpec((1, 1, _SEQ), lambda i: (i, 0, 0)),
            pl.BlockSpec((1, 1), lambda i: (0, 0)),
        ],
        out_shape=[
            jax.ShapeDtypeStruct((_BATCH, _SEQ, _DIM), jnp.float32),
            jax.ShapeDtypeStruct((_BATCH, 1, _SEQ), jnp.int32),
            jax.ShapeDtypeStruct((1, 1), jnp.float32),
        ],
        compiler_params=pltpu.CompilerParams(
            dimension_semantics=("arbitrary",),
        ),
    )(inputs, W, w2)

    mse = acc[0, 0] / jnp.float32(_ROWS * _DIM)
    loss = mse + 0.25 * mse
    return (loss, qst, idx.reshape(_ROWS, 1))


# speedup vs baseline: 1.1686x; 1.1686x over previous
import functools

import jax
import jax.numpy as jnp
from jax.experimental import pallas as pl
from jax.experimental.pallas import tpu as pltpu

_NUM_EMB = 1024
_DIM = 64
_BATCH = 64
_SEQ = 576
_ROWS = _BATCH * _SEQ
_BPG = 4
_BLK = _BPG * _SEQ
_NBLK = _BATCH // _BPG


def _vq_block_kernel(x_ref, w_ref, w2_ref, qst_ref, idx_ref, acc_ref):
    x = x_ref[...].reshape(_BLK, _DIM)
    w = w_ref[...]
    s = jax.lax.dot_general(x, w, (((1,), (1,)), ((), ())),
                            preferred_element_type=jnp.float32)
    x2 = jnp.sum(x * x, axis=1, keepdims=True)
    d = (x2 + w2_ref[...]) - 2.0 * s
    m = jnp.min(d, axis=1, keepdims=True)
    lane = jax.lax.broadcasted_iota(jnp.int32, (_BLK, _NUM_EMB), 1)
    idx = jnp.min(jnp.where(d == m, lane, _NUM_EMB), axis=1)
    enc = (lane == idx[:, None]).astype(jnp.bfloat16)
    q = jax.lax.dot_general(enc, w.astype(jnp.bfloat16),
                            (((1,), (0,)), ((), ())),
                            preferred_element_type=jnp.float32)
    qst_ref[...] = (x + (q - x)).reshape(_BPG, _SEQ, _DIM)
    idx_ref[...] = idx[None, None, :]
    part = jnp.sum((q - x) ** 2)

    @pl.when(pl.program_id(0) == 0)
    def _init():
        acc_ref[...] = jnp.zeros_like(acc_ref)

    acc_ref[...] += part


@functools.partial(jax.jit, static_argnames=())
def kernel(inputs, W):
    w2 = jnp.sum(W ** 2, axis=1).reshape(1, _NUM_EMB)

    qst, idx, acc = pl.pallas_call(
        _vq_block_kernel,
        grid=(_NBLK,),
        in_specs=[
            pl.BlockSpec((_BPG, _SEQ, _DIM), lambda i: (i, 0, 0)),
            pl.BlockSpec((_NUM_EMB, _DIM), lambda i: (0, 0)),
            pl.BlockSpec((1, _NUM_EMB), lambda i: (0, 0)),
        ],
        out_specs=[
            pl.BlockSpec((_BPG, _SEQ, _DIM), lambda i: (i, 0, 0)),
            pl.BlockSpec((1, 1, _BLK), lambda i: (i, 0, 0)),
            pl.BlockSpec((1, 1), lambda i: (0, 0)),
        ],
        out_shape=[
            jax.ShapeDtypeStruct((_BATCH, _SEQ, _DIM), jnp.float32),
            jax.ShapeDtypeStruct((_NBLK, 1, _BLK), jnp.int32),
            jax.ShapeDtypeStruct((1, 1), jnp.float32),
        ],
        compiler_params=pltpu.CompilerParams(
            dimension_semantics=("arbitrary",),
        ),
    )(inputs, W, w2)

    mse = acc[0, 0] / jnp.float32(_ROWS * _DIM)
    loss = mse + 0.25 * mse
    return (loss, qst, idx.reshape(_ROWS, 1))


# speedup vs baseline: 1.1981x; 1.0252x over previous
import functools

import jax
import jax.numpy as jnp
from jax.experimental import pallas as pl
from jax.experimental.pallas import tpu as pltpu

_NUM_EMB = 1024
_DIM = 64
_BATCH = 64
_SEQ = 576
_ROWS = _BATCH * _SEQ
_BPG = 8
_BLK = _BPG * _SEQ
_NBLK = _BATCH // _BPG


def _vq_block_kernel(x_ref, w_ref, w2_ref, qst_ref, idx_ref, acc_ref):
    x = x_ref[...].reshape(_BLK, _DIM)
    w = w_ref[...]
    s = jax.lax.dot_general(x, w, (((1,), (1,)), ((), ())),
                            preferred_element_type=jnp.float32)
    x2 = jnp.sum(x * x, axis=1, keepdims=True)
    d = (x2 + w2_ref[...]) - 2.0 * s
    m = jnp.min(d, axis=1, keepdims=True)
    lane = jax.lax.broadcasted_iota(jnp.int32, (_BLK, _NUM_EMB), 1)
    idx = jnp.min(jnp.where(d == m, lane, _NUM_EMB), axis=1)
    enc = (lane == idx[:, None]).astype(jnp.bfloat16)
    q = jax.lax.dot_general(enc, w.astype(jnp.bfloat16),
                            (((1,), (0,)), ((), ())),
                            preferred_element_type=jnp.float32)
    qst_ref[...] = (x + (q - x)).reshape(_BPG, _SEQ, _DIM)
    idx_ref[...] = idx[None, None, :]
    part = jnp.sum((q - x) ** 2)

    @pl.when(pl.program_id(0) == 0)
    def _init():
        acc_ref[...] = jnp.zeros_like(acc_ref)

    acc_ref[...] += part


@functools.partial(jax.jit, static_argnames=())
def kernel(inputs, W):
    w2 = jnp.sum(W ** 2, axis=1).reshape(1, _NUM_EMB)

    qst, idx, acc = pl.pallas_call(
        _vq_block_kernel,
        grid=(_NBLK,),
        in_specs=[
            pl.BlockSpec((_BPG, _SEQ, _DIM), lambda i: (i, 0, 0)),
            pl.BlockSpec((_NUM_EMB, _DIM), lambda i: (0, 0)),
            pl.BlockSpec((1, _NUM_EMB), lambda i: (0, 0)),
        ],
        out_specs=[
            pl.BlockSpec((_BPG, _SEQ, _DIM), lambda i: (i, 0, 0)),
            pl.BlockSpec((1, 1, _BLK), lambda i: (i, 0, 0)),
            pl.BlockSpec((1, 1), lambda i: (0, 0)),
        ],
        out_shape=[
            jax.ShapeDtypeStruct((_BATCH, _SEQ, _DIM), jnp.float32),
            jax.ShapeDtypeStruct((_NBLK, 1, _BLK), jnp.int32),
            jax.ShapeDtypeStruct((1, 1), jnp.float32),
        ],
        compiler_params=pltpu.CompilerParams(
            dimension_semantics=("arbitrary",),
        ),
    )(inputs, W, w2)

    mse = acc[0, 0] / jnp.float32(_ROWS * _DIM)
    loss = mse + 0.25 * mse
    return (loss, qst, idx.reshape(_ROWS, 1))


# speedup vs baseline: 1.3227x; 1.1040x over previous
import functools

import jax
import jax.numpy as jnp
from jax.experimental import pallas as pl
from jax.experimental.pallas import tpu as pltpu

_NUM_EMB = 1024
_DIM = 64
_BATCH = 64
_SEQ = 576
_ROWS = _BATCH * _SEQ
_BPG = 8
_BLK = _BPG * _SEQ
_NBLK = _BATCH // _BPG


def _vq_block_kernel(x_ref, w_ref, w2_ref, qst_ref, idx_ref, acc_ref):
    x = x_ref[...].reshape(_BLK, _DIM)
    w = w_ref[...]
    s = jax.lax.dot_general(x, w, (((1,), (1,)), ((), ())),
                            preferred_element_type=jnp.float32)
    x2 = jnp.sum(x * x, axis=1, keepdims=True)
    d = (x2 + w2_ref[...]) - 2.0 * s
    lane = jax.lax.broadcasted_iota(jnp.int32, (_BLK, _NUM_EMB), 1)
    idx = jnp.argmin(d, axis=1).astype(jnp.int32)
    enc = (lane == idx[:, None]).astype(jnp.bfloat16)
    q = jax.lax.dot_general(enc, w.astype(jnp.bfloat16),
                            (((1,), (0,)), ((), ())),
                            preferred_element_type=jnp.float32)
    qst_ref[...] = (x + (q - x)).reshape(_BPG, _SEQ, _DIM)
    idx_ref[...] = idx[None, None, :]
    part = jnp.sum((q - x) ** 2)

    @pl.when(pl.program_id(0) == 0)
    def _init():
        acc_ref[...] = jnp.zeros_like(acc_ref)

    acc_ref[...] += part


@functools.partial(jax.jit, static_argnames=())
def kernel(inputs, W):
    w2 = jnp.sum(W ** 2, axis=1).reshape(1, _NUM_EMB)

    qst, idx, acc = pl.pallas_call(
        _vq_block_kernel,
        grid=(_NBLK,),
        in_specs=[
            pl.BlockSpec((_BPG, _SEQ, _DIM), lambda i: (i, 0, 0)),
            pl.BlockSpec((_NUM_EMB, _DIM), lambda i: (0, 0)),
            pl.BlockSpec((1, _NUM_EMB), lambda i: (0, 0)),
        ],
        out_specs=[
            pl.BlockSpec((_BPG, _SEQ, _DIM), lambda i: (i, 0, 0)),
            pl.BlockSpec((1, 1, _BLK), lambda i: (i, 0, 0)),
            pl.BlockSpec((1, 1), lambda i: (0, 0)),
        ],
        out_shape=[
            jax.ShapeDtypeStruct((_BATCH, _SEQ, _DIM), jnp.float32),
            jax.ShapeDtypeStruct((_NBLK, 1, _BLK), jnp.int32),
            jax.ShapeDtypeStruct((1, 1), jnp.float32),
        ],
        compiler_params=pltpu.CompilerParams(
            dimension_semantics=("arbitrary",),
        ),
    )(inputs, W, w2)

    mse = acc[0, 0] / jnp.float32(_ROWS * _DIM)
    loss = mse + 0.25 * mse
    return (loss, qst, idx.reshape(_ROWS, 1))
